# baseline (speedup 1.0000x reference)
import functools

import jax
import jax.numpy as jnp
from jax import lax
from jax.experimental import pallas as pl
from jax.experimental.pallas import tpu as pltpu
from jax.experimental.pallas import tpu_sc as plsc

_DIM = 64
_NUM_WORKERS = 32
_CHUNK = 128
_NS = 4
_LANES = 16


def _build_body(n_t):
  def body(xt_hbm, emb_hbm, out_hbm, idx_v, gbuf, tbuf, stsem, gsems, ssems):
    c = lax.axis_index("c")
    s = lax.axis_index("s")
    wid = s * 2 + c
    col0 = wid * _CHUNK

    def stage(t, _):
      pltpu.make_async_copy(
          xt_hbm.at[t, pl.ds(col0, _CHUNK)], idx_v.at[t], stsem).start()
      return 0

    lax.fori_loop(0, n_t, stage, 0)

    def drain_stage(t, _):
      pltpu.make_async_copy(
          xt_hbm.at[0, pl.ds(col0, _CHUNK)], idx_v.at[0], stsem).wait()
      return 0

    lax.fori_loop(0, n_t, drain_stage, 0)

    def start_gather(t, b):
      pltpu.make_async_copy(
          emb_hbm.at[idx_v.at[t]], gbuf.at[b], gsems.at[b]).start()

    def wait_gather(b):
      pltpu.make_async_copy(
          emb_hbm.at[idx_v.at[0]], gbuf.at[b], gsems.at[b]).wait()

    def wait_store8(b):
      for _ in range(8):
        pltpu.make_async_copy(
            tbuf.at[0, pl.ds(0, 8)], out_hbm.at[0, 0, wid], ssems.at[b]).wait()

    bidx = [lax.iota(jnp.int32, _LANES) + _LANES * k for k in range(8)]

    def prime(t, _):
      start_gather(t, t)
      return 0

    lax.fori_loop(0, _NS, prime, 0)

    def step(t, _):
      b = lax.rem(t, _NS)
      bv = jnp.full((_LANES,), 0, jnp.int32) + b
      wait_gather(b)

      @pl.when(t >= _NS)
      def _():
        wait_store8(b)

      def band(d_hi, _):
        for d_lo in range(8):
          d = d_hi * 8 + d_lo
          dv = jnp.full((_LANES,), 0, jnp.int32) + d
          for k in range(8):
            tbuf[b, d, pl.ds(_LANES * k, _LANES)] = plsc.load_gather(
                gbuf, [bv, bidx[k], dv])
        pltpu.make_async_copy(
            tbuf.at[b, pl.ds(d_hi * 8, 8)], out_hbm.at[t, d_hi, wid],
            ssems.at[b]).start()
        return 0

      lax.fori_loop(0, 8, band, 0)

      @pl.when(t + _NS < n_t)
      def _():
        start_gather(t + _NS, b)

      return 0

    lax.fori_loop(0, n_t, step, 0)

    def drain(b, _):
      wait_store8(b)
      return 0

    lax.fori_loop(0, _NS, drain, 0)

  return body


@functools.partial(jax.jit, static_argnums=())
def _embed(xt, embedding):
  n_t, n_b = xt.shape
  mesh = plsc.VectorSubcoreMesh(core_axis_name="c", subcore_axis_name="s")
  kfn = pl.kernel(
      _build_body(n_t),
      out_type=jax.ShapeDtypeStruct(
          (n_t, 8, _NUM_WORKERS, 8, _CHUNK), jnp.float32),
      mesh=mesh,
      scratch_types=[
          pltpu.VMEM((n_t, _CHUNK), jnp.int32),
          pltpu.VMEM((_NS, _CHUNK, _DIM), jnp.float32),
          pltpu.VMEM((_NS, _DIM, _CHUNK), jnp.float32),
          pltpu.SemaphoreType.DMA,
          pltpu.SemaphoreType.DMA((_NS,)),
          pltpu.SemaphoreType.DMA((_NS,)),
      ],
      compiler_params=pltpu.CompilerParams(
          use_tc_tiling_on_sc=False, needs_layout_passes=False),
  )
  return kfn(xt, embedding)


def kernel(x, embedding):
  b, t = x.shape
  out5 = _embed(x.T, embedding)
  return out5.transpose(2, 4, 0, 1, 3).reshape(b, t, _DIM)


# speedup vs baseline: 1.5593x; 1.5593x over previous
import functools

import jax
import jax.numpy as jnp
from jax import lax
from jax.experimental import pallas as pl
from jax.experimental.pallas import tpu as pltpu
from jax.experimental.pallas import tpu_sc as plsc

_DIM = 64
_NUM_WORKERS = 32
_CHUNK = 128
_NS = 8
_H = 4


def _build_body(n_t):
  def body(xt_hbm, emb_hbm, out_hbm, idx_v, bufs, stsem, gsems, ssems):
    c = lax.axis_index("c")
    s = lax.axis_index("s")
    wid = s * 2 + c
    col0 = wid * _CHUNK

    def stage(t, _):
      pltpu.make_async_copy(
          xt_hbm.at[t, pl.ds(col0, _CHUNK)], idx_v.at[t], stsem).start()
      return 0

    lax.fori_loop(0, n_t, stage, 0)

    def drain_stage(t, _):
      pltpu.make_async_copy(
          xt_hbm.at[0, pl.ds(col0, _CHUNK)], idx_v.at[0], stsem).wait()
      return 0

    lax.fori_loop(0, n_t, drain_stage, 0)

    def start_gather(t, b):
      pltpu.make_async_copy(
          emb_hbm.at[idx_v.at[t]], bufs[b], gsems[b]).start()

    def wait_gather(b):
      pltpu.make_async_copy(
          emb_hbm.at[idx_v.at[0]], bufs[b], gsems[b]).wait()

    def start_store(t, b):
      pltpu.make_async_copy(bufs[b], out_hbm.at[t, wid], ssems[b]).start()

    def wait_store(b):
      pltpu.make_async_copy(bufs[b], out_hbm.at[0, wid], ssems[b]).wait()

    for t in range(_H):
      start_gather(t, t % _NS)

    for b in range(_NS):
      tg = b + _H
      if tg >= _NS:
        wait_store(tg % _NS)
      start_gather(tg, tg % _NS)
      wait_gather(b)
      start_store(b, b)

    def outer(k, _):
      t0 = k * _NS
      for b in range(_NS):
        t = t0 + b
        bg = (b + _H) % _NS
        wait_store(bg)
        start_gather(t + _H, bg)
        wait_gather(b)
        start_store(t, b)
      return 0

    lax.fori_loop(1, n_t // _NS - 1, outer, 0)

    for b in range(_NS):
      t = n_t - _NS + b
      tg = t + _H
      if tg < n_t:
        bg = tg % _NS
        wait_store(bg)
        start_gather(tg, bg)
      wait_gather(b)
      start_store(t, b)

    for b in range(_NS):
      wait_store(b)

  return body


@functools.partial(jax.jit, static_argnums=())
def _embed(xt, embedding):
  n_t, n_b = xt.shape
  mesh = plsc.VectorSubcoreMesh(core_axis_name="c", subcore_axis_name="s")
  kfn = pl.kernel(
      _build_body(n_t),
      out_type=jax.ShapeDtypeStruct(
          (n_t, _NUM_WORKERS, _CHUNK, _DIM), jnp.float32),
      mesh=mesh,
      scratch_types=[
          pltpu.VMEM((n_t, _CHUNK), jnp.int32),
          [pltpu.VMEM((_CHUNK, _DIM), jnp.float32) for _ in range(_NS)],
          pltpu.SemaphoreType.DMA,
          [pltpu.SemaphoreType.DMA for _ in range(_NS)],
          [pltpu.SemaphoreType.DMA for _ in range(_NS)],
      ],
      compiler_params=pltpu.CompilerParams(use_tc_tiling_on_sc=False),
  )
  return kfn(xt, embedding)


def kernel(x, embedding):
  b, t = x.shape
  out = _embed(x.T, embedding)
  return out.transpose(1, 2, 0, 3).reshape(b, t, _DIM)
